# baseline (speedup 1.0000x reference)
import functools

import jax
import jax.numpy as jnp
from jax import lax
from jax.experimental import pallas as pl
from jax.experimental.pallas import tpu as pltpu
from jax.experimental.pallas import tpu_sc as plsc


def _sc_gather(table2, idx, num_cores, num_subcores, nbuf):
    n = idx.shape[0]
    nw = num_cores * num_subcores
    nb_blocks = 16384 // 128
    units = n // 128
    per_w = units // nw
    mesh = plsc.VectorSubcoreMesh(core_axis_name="c", subcore_axis_name="s")

    @functools.partial(
        pl.kernel,
        mesh=mesh,
        out_type=jax.ShapeDtypeStruct((50, 8, 128, 8, 128), jnp.float32),
        scratch_types=[
            pltpu.VMEM((per_w * 128,), jnp.int32),
            [pltpu.VMEM((128, 64), jnp.float32) for _ in range(nbuf)],
            [pltpu.VMEM((8, 8, 128), jnp.float32) for _ in range(nbuf)],
            [pltpu.SemaphoreType.DMA for _ in range(nbuf)],
            [pltpu.SemaphoreType.DMA for _ in range(nbuf)],
        ],
        compiler_params=pltpu.CompilerParams(
            use_tc_tiling_on_sc=False, needs_layout_passes=False
        ),
    )
    def k(idx_hbm, table_hbm, out_hbm, idx_v, rows_v, outt_v, sem_g, sem_s):
        wid = lax.axis_index("s") * num_cores + lax.axis_index("c")
        base_u = wid * per_w
        pltpu.sync_copy(idx_hbm.at[pl.ds(base_u * 128, per_w * 128)], idx_v)

        def gather(b, u):
            uc = jnp.minimum(u, per_w - 1)
            return pltpu.make_async_copy(
                table_hbm.at[idx_v.at[pl.ds(uc * 128, 128)]],
                rows_v[b],
                sem_g[b],
            )

        def store(b, u):
            gu = base_u + jnp.minimum(u, per_w - 1)
            h = gu // nb_blocks
            j = gu % nb_blocks
            return pltpu.make_async_copy(
                outt_v[b], out_hbm.at[h, :, j], sem_s[b]
            )

        tvec = lax.iota(jnp.int32, 16)

        def transpose(b):
            @pl.loop(0, 64)
            def _(cc):
                ci = cc // 8
                cs = cc % 8
                cvec = jnp.full((16,), cc, jnp.int32)
                for kk in range(8):
                    v = plsc.load_gather(rows_v[b], [tvec + 16 * kk, cvec])
                    outt_v[b][ci, cs, pl.ds(16 * kk, 16)] = v

        for b in range(nbuf):
            gather(b, b).start()

        @pl.loop(0, per_w, step=nbuf)
        def _(u0):
            for b in range(nbuf):
                u = u0 + b

                @pl.when(u0 > 0)
                def _():
                    store(b, 0).wait()

                gather(b, u).wait()
                transpose(b)
                store(b, u).start()
                gather(b, u + nbuf).start()

        for b in range(nbuf):
            gather(b, 0).wait()
            store(b, 0).wait()

    return k(idx, table2)


def kernel(tokens, table):
    bsz, h = tokens.shape
    d = table.shape[1]
    idx = tokens.T.reshape(bsz * h).astype(jnp.int32)
    info = plsc.get_sparse_core_info()
    out5 = _sc_gather(table, idx, info.num_cores, info.num_subcores, 4)
    return out5.transpose(2, 4, 0, 1, 3).reshape(bsz, h, d)


# speedup vs baseline: 1.4565x; 1.4565x over previous
import functools

import jax
import jax.numpy as jnp
from jax import lax
from jax.experimental import pallas as pl
from jax.experimental.pallas import tpu as pltpu
from jax.experimental.pallas import tpu_sc as plsc


def _sc_gather(table2, idx, num_cores, num_subcores, nbuf):
    n = idx.shape[0]
    nw = num_cores * num_subcores
    nb_blocks = 16384 // 128
    units = n // 128
    per_w = units // nw
    mesh = plsc.VectorSubcoreMesh(core_axis_name="c", subcore_axis_name="s")

    @functools.partial(
        pl.kernel,
        mesh=mesh,
        out_type=jax.ShapeDtypeStruct((50, 8, 128, 8, 128), jnp.float32),
        scratch_types=[
            pltpu.VMEM((per_w * 128,), jnp.int32),
            [pltpu.VMEM((128, 64), jnp.float32) for _ in range(nbuf)],
            [pltpu.VMEM((8, 8, 128), jnp.float32) for _ in range(nbuf)],
            [pltpu.SemaphoreType.DMA for _ in range(nbuf)],
            [pltpu.SemaphoreType.DMA for _ in range(nbuf)],
        ],
        compiler_params=pltpu.CompilerParams(
            use_tc_tiling_on_sc=False, needs_layout_passes=False
        ),
    )
    def k(idx_hbm, table_hbm, out_hbm, idx_v, rows_v, outt_v, sem_g, sem_s):
        wid = lax.axis_index("s") * num_cores + lax.axis_index("c")
        base_u = wid * per_w
        pltpu.sync_copy(idx_hbm.at[pl.ds(base_u * 128, per_w * 128)], idx_v)

        def gather(b, u):
            uc = jnp.minimum(u, per_w - 1)
            return pltpu.make_async_copy(
                table_hbm.at[idx_v.at[pl.ds(uc * 128, 128)]],
                rows_v[b],
                sem_g[b],
            )

        def store(b, u):
            gu = base_u + jnp.minimum(u, per_w - 1)
            h = gu // nb_blocks
            j = gu % nb_blocks
            return pltpu.make_async_copy(
                outt_v[b], out_hbm.at[h, :, j], sem_s[b]
            )

        tvecs = [lax.iota(jnp.int32, 16) + 16 * kk for kk in range(8)]

        def transpose(b):
            @plsc.parallel_loop(0, 64, unroll=4)
            def _(cc):
                ci = cc // 8
                cs = cc % 8
                cvec = jnp.full((16,), cc, jnp.int32)
                for kk in range(8):
                    v = plsc.load_gather(rows_v[b], [tvecs[kk], cvec])
                    outt_v[b][ci, cs, pl.ds(16 * kk, 16)] = v

        for b in range(nbuf):
            gather(b, b).start()

        @pl.loop(0, per_w, step=nbuf)
        def _(u0):
            for b in range(nbuf):
                u = u0 + b

                @pl.when(u0 > 0)
                def _():
                    store(b, 0).wait()

                gather(b, u).wait()
                transpose(b)
                store(b, u).start()
                gather(b, u + nbuf).start()

        for b in range(nbuf):
            gather(b, 0).wait()
            store(b, 0).wait()

    return k(idx, table2)


def kernel(tokens, table):
    bsz, h = tokens.shape
    d = table.shape[1]
    idx = tokens.T.reshape(bsz * h).astype(jnp.int32)
    info = plsc.get_sparse_core_info()
    out5 = _sc_gather(table, idx, info.num_cores, info.num_subcores, 4)
    return out5.transpose(2, 4, 0, 1, 3).reshape(bsz, h, d)


# speedup vs baseline: 2.4089x; 1.6539x over previous
import functools

import jax
import jax.numpy as jnp
from jax import lax
from jax.experimental import pallas as pl
from jax.experimental.pallas import tpu as pltpu
from jax.experimental.pallas import tpu_sc as plsc


def _sc_gather(table2, idx, num_cores, num_subcores, nbuf):
    n = idx.shape[0]
    nw = num_cores * num_subcores
    nb_blocks = 16384 // 128
    units = n // 128
    per_w = units // nw
    mesh = plsc.VectorSubcoreMesh(core_axis_name="c", subcore_axis_name="s")

    @functools.partial(
        pl.kernel,
        mesh=mesh,
        out_type=jax.ShapeDtypeStruct((50, 8, 128, 8, 128), jnp.float32),
        scratch_types=[
            pltpu.VMEM((per_w * 128,), jnp.int32),
            [pltpu.VMEM((128, 64), jnp.float32) for _ in range(nbuf)],
            [pltpu.VMEM((8, 8, 137), jnp.float32) for _ in range(nbuf)],
            [pltpu.SemaphoreType.DMA for _ in range(nbuf)],
            [pltpu.SemaphoreType.DMA for _ in range(nbuf)],
        ],
        compiler_params=pltpu.CompilerParams(
            use_tc_tiling_on_sc=False, needs_layout_passes=False
        ),
    )
    def k(idx_hbm, table_hbm, out_hbm, idx_v, rows_v, outt_v, sem_g, sem_s):
        wid = lax.axis_index("s") * num_cores + lax.axis_index("c")
        base_u = wid * per_w
        pltpu.sync_copy(idx_hbm.at[pl.ds(base_u * 128, per_w * 128)], idx_v)

        def gather(b, u):
            uc = jnp.minimum(u, per_w - 1)
            return pltpu.make_async_copy(
                table_hbm.at[idx_v.at[pl.ds(uc * 128, 128)]],
                rows_v[b],
                sem_g[b],
            )

        def store(b, u):
            gu = base_u + jnp.minimum(u, per_w - 1)
            h = gu // nb_blocks
            j = gu % nb_blocks
            return pltpu.make_async_copy(
                outt_v[b].at[:, :, pl.ds(0, 128)], out_hbm.at[h, :, j], sem_s[b]
            )

        lane = lax.iota(jnp.int32, 16)
        ci_q = [(lane + 16 * q) // 8 for q in range(4)]
        cs_q = [(lane + 16 * q) % 8 for q in range(4)]

        def transpose(b):
            @plsc.parallel_loop(0, 128, unroll=4)
            def _(t):
                tb = jnp.full((16,), t, jnp.int32)
                for q in range(4):
                    v = rows_v[b][t, pl.ds(16 * q, 16)]
                    plsc.store_scatter(outt_v[b], [ci_q[q], cs_q[q], tb], v)

        for b in range(nbuf):
            gather(b, b).start()

        @pl.loop(0, per_w, step=nbuf)
        def _(u0):
            for b in range(nbuf):
                u = u0 + b

                @pl.when(u0 > 0)
                def _():
                    store(b, 0).wait()

                gather(b, u).wait()
                transpose(b)
                store(b, u).start()
                gather(b, u + nbuf).start()

        for b in range(nbuf):
            gather(b, 0).wait()
            store(b, 0).wait()

    return k(idx, table2)


def kernel(tokens, table):
    bsz, h = tokens.shape
    d = table.shape[1]
    idx = tokens.T.reshape(bsz * h).astype(jnp.int32)
    info = plsc.get_sparse_core_info()
    out5 = _sc_gather(table, idx, info.num_cores, info.num_subcores, 4)
    return out5.transpose(2, 4, 0, 1, 3).reshape(bsz, h, d)
